# baseline (speedup 1.0000x reference)
import functools

import jax
import jax.numpy as jnp
import numpy as np
from jax.experimental import pallas as pl
from jax.experimental.pallas import tpu as pltpu

D_MODEL = 1024
N_HEADS = 16
D_HEAD = 64
N_EXP = 8
ROPE_BASE = 10000.0
NEG_INF = -1e30
HD = N_HEADS * D_HEAD
EHD = N_EXP * HD
EH = N_EXP * N_HEADS


def _top2_gates(logits_planes):
    m1 = logits_planes[0]
    i1 = jnp.zeros_like(m1)
    for e in range(1, N_EXP):
        gt = logits_planes[e] > m1
        m1 = jnp.where(gt, logits_planes[e], m1)
        i1 = jnp.where(gt, float(e), i1)
    p0 = jnp.where(i1 == 0.0, NEG_INF, logits_planes[0])
    m2 = p0
    i2 = jnp.zeros_like(m1)
    for e in range(1, N_EXP):
        pe = jnp.where(i1 == float(e), NEG_INF, logits_planes[e])
        gt = pe > m2
        m2 = jnp.where(gt, pe, m2)
        i2 = jnp.where(gt, float(e), i2)
    g1 = jax.nn.sigmoid(m1)
    g2 = jax.nn.sigmoid(m2)
    return [jnp.where(i1 == float(e), g1, 0.0) + jnp.where(i2 == float(e), g2, 0.0)
            for e in range(N_EXP)]


def _dot(a, b, trans_b=False):
    dims = (((1,), (1 if trans_b else 0,)), ((), ()))
    return jax.lax.dot_general(a, b, dims, preferred_element_type=jnp.float32)


def _swap_halves_lanes(t):
    r1 = pltpu.roll(t, HD - 32, 1)
    r2 = pltpu.roll(t, 32, 1)
    lane = jax.lax.broadcasted_iota(jnp.int32, t.shape, 1)
    return jnp.where(lane % D_HEAD < D_HEAD // 2, r1, r2)


def _proj_kernel(x_ref, wq4_ref, wv_ref, wsel_ref, rep_ref, cos_ref, sin_ref,
                 q_ref, k_ref, v_ref, go_ref):
    x = x_ref[...]
    xb = x.astype(jnp.bfloat16)

    qk4 = _dot(xb, wq4_ref[...])
    C = cos_ref[...]
    Sg = sin_ref[...]
    qh = qk4[:, :HD]
    kh = qk4[:, HD:]
    q_ref[...] = (qh * C + _swap_halves_lanes(qh) * Sg).astype(jnp.bfloat16)
    k_ref[...] = (kh * C + _swap_halves_lanes(kh) * Sg).astype(jnp.bfloat16)

    lg = _dot(x, wsel_ref[...])
    planes_v = [lg[:, e * N_HEADS:(e + 1) * N_HEADS] for e in range(N_EXP)]
    planes_o = [lg[:, EH + e * N_HEADS:EH + (e + 1) * N_HEADS] for e in range(N_EXP)]
    gates_v = _top2_gates(planes_v)
    gates_o = _top2_gates(planes_o)
    go_ref[...] = jnp.concatenate(gates_o, axis=1)
    gvb = jnp.concatenate(gates_v, axis=1).astype(jnp.bfloat16)

    vacc = None
    for e in range(N_EXP):
        ge = _dot(gvb, rep_ref[:, e * HD:(e + 1) * HD])
        ve = _dot(xb, wv_ref[:, e * HD:(e + 1) * HD])
        term = ve * ge
        vacc = term if vacc is None else vacc + term
    v_ref[...] = vacc.astype(jnp.bfloat16)


def _attn_kernel(q_ref, k_ref, v_ref, go_ref, rep_ref, wo_ref, o_ref, *, tq, tk):
    qt = pl.program_id(0)
    dh = D_HEAD
    rows = jax.lax.broadcasted_iota(jnp.int32, (tq, tk), 0) + qt * tq
    cols0 = jax.lax.broadcasted_iota(jnp.int32, (tq, tk), 1)
    ones = jnp.ones((tk, dh), jnp.bfloat16)
    m0 = jnp.full((tq, 1), NEG_INF, jnp.float32)
    a0 = jnp.zeros((tq, 2 * dh), jnp.float32)

    def step(q_, h, j, m, accl, masked):
        kj = k_ref[pl.ds(j * tk, tk), h * dh:(h + 1) * dh]
        vj = v_ref[pl.ds(j * tk, tk), h * dh:(h + 1) * dh]
        s = _dot(q_, kj, trans_b=True)
        if masked:
            s = jnp.where(cols0 + j * tk <= rows, s, NEG_INF)
        mn = jnp.maximum(m, jnp.max(s, axis=-1, keepdims=True))
        alpha = jnp.exp(m - mn)
        p = jnp.exp(s - mn)
        vjx = jnp.concatenate([vj, ones], axis=1)
        accl2 = accl * alpha + _dot(p.astype(jnp.bfloat16), vjx)
        return mn, accl2

    ctx_list = [None] * N_HEADS
    for h0 in range(0, N_HEADS, 2):
        qa = q_ref[:, h0 * dh:(h0 + 1) * dh]
        qb = q_ref[:, (h0 + 1) * dh:(h0 + 2) * dh]

        def body(j, carry, h0=h0, qa=qa, qb=qb):
            ma, accla, mb, acclb = carry
            ma, accla = step(qa, h0, j, ma, accla, masked=False)
            mb, acclb = step(qb, h0 + 1, j, mb, acclb, masked=False)
            return ma, accla, mb, acclb

        nfull = (qt * tq) // tk
        ma, accla, mb, acclb = jax.lax.fori_loop(0, nfull, body, (m0, a0, m0, a0))
        ma, accla = step(qa, h0, nfull, ma, accla, masked=True)
        mb, acclb = step(qb, h0 + 1, nfull, mb, acclb, masked=True)
        ctx_list[h0] = accla[:, :dh] / accla[:, dh:dh + 1]
        ctx_list[h0 + 1] = acclb[:, :dh] / acclb[:, dh:dh + 1]
    ctx = jnp.concatenate(ctx_list, axis=1)
    gob = go_ref[...].astype(jnp.bfloat16)
    acc = None
    for e in range(N_EXP):
        ge = _dot(gob, rep_ref[:, e * HD:(e + 1) * HD])
        blk = (ctx * ge).astype(jnp.bfloat16)
        pe = _dot(blk, wo_ref[e * HD:(e + 1) * HD, :])
        acc = pe if acc is None else acc + pe
    o_ref[...] = acc


def kernel(token_stream, Wq, Wk, Wv, Wo, Wsel_v, Wsel_o):
    B, S, _ = token_stream.shape
    x = token_stream.reshape(S, D_MODEL)
    H, E, dh = N_HEADS, N_EXP, D_HEAD
    half = dh // 2
    f32 = jnp.float32
    bf16 = jnp.bfloat16

    scale = 1.0 / np.sqrt(dh)
    wq4 = jnp.concatenate([Wq * scale, Wk], axis=1).astype(bf16)
    wv_em = Wv.transpose(2, 1, 0, 3).reshape(D_MODEL, EHD).astype(bf16)
    wo_em = Wo.transpose(1, 0, 2, 3).reshape(EHD, D_MODEL).astype(bf16)
    wsel = jnp.concatenate(
        [Wsel_v.reshape(D_MODEL, H, E).transpose(0, 2, 1).reshape(D_MODEL, EH),
         Wsel_o.reshape(D_MODEL, H, E).transpose(0, 2, 1).reshape(D_MODEL, EH)],
        axis=1)

    rcols = jnp.arange(EHD)
    e_c = rcols // HD
    h_c = (rcols % HD) // dh
    rep = (jnp.arange(EH)[:, None] == (e_c * H + h_c)[None, :]).astype(bf16)

    pos = jnp.arange(S, dtype=f32)
    inv_freq = 1.0 / (ROPE_BASE ** (jnp.arange(0, dh, 2, dtype=f32) / dh))
    ang = pos[:, None] * inv_freq[None, :]
    chead = jnp.concatenate([jnp.cos(ang), jnp.cos(ang)], axis=1)
    shead = jnp.concatenate([-jnp.sin(ang), jnp.sin(ang)], axis=1)
    ctab = jnp.tile(chead, (1, H))
    stab = jnp.tile(shead, (1, H))

    T = min(256, S)
    nt = S // T

    q, k, v, go = pl.pallas_call(
        _proj_kernel,
        grid=(nt,),
        in_specs=[
            pl.BlockSpec((T, D_MODEL), lambda t: (t, 0)),
            pl.BlockSpec((D_MODEL, 2 * HD), lambda t: (0, 0)),
            pl.BlockSpec((D_MODEL, EHD), lambda t: (0, 0)),
            pl.BlockSpec((D_MODEL, 2 * EH), lambda t: (0, 0)),
            pl.BlockSpec((EH, EHD), lambda t: (0, 0)),
            pl.BlockSpec((T, HD), lambda t: (t, 0)),
            pl.BlockSpec((T, HD), lambda t: (t, 0)),
        ],
        out_specs=[
            pl.BlockSpec((T, HD), lambda t: (t, 0)),
            pl.BlockSpec((T, HD), lambda t: (t, 0)),
            pl.BlockSpec((T, HD), lambda t: (t, 0)),
            pl.BlockSpec((T, EH), lambda t: (t, 0)),
        ],
        out_shape=[
            jax.ShapeDtypeStruct((S, HD), bf16),
            jax.ShapeDtypeStruct((S, HD), bf16),
            jax.ShapeDtypeStruct((S, HD), bf16),
            jax.ShapeDtypeStruct((S, EH), f32),
        ],
    )(x, wq4, wv_em, wsel, rep, ctab, stab)

    Tq = min(512, S)
    Tk = min(1024, S)
    nq = S // Tq
    out = pl.pallas_call(
        functools.partial(_attn_kernel, tq=Tq, tk=Tk),
        grid=(nq,),
        in_specs=[
            pl.BlockSpec((Tq, HD), lambda qt: (qt, 0)),
            pl.BlockSpec((S, HD), lambda qt: (0, 0)),
            pl.BlockSpec((S, HD), lambda qt: (0, 0)),
            pl.BlockSpec((Tq, EH), lambda qt: (qt, 0)),
            pl.BlockSpec((EH, EHD), lambda qt: (0, 0)),
            pl.BlockSpec((EHD, D_MODEL), lambda qt: (0, 0)),
        ],
        out_specs=pl.BlockSpec((Tq, D_MODEL), lambda qt: (qt, 0)),
        out_shape=jax.ShapeDtypeStruct((S, D_MODEL), f32),
    )(q, k, v, go, rep, wo_em)

    return out.reshape(B, S, D_MODEL)


# speedup vs baseline: 1.1352x; 1.1352x over previous
import functools

import jax
import jax.numpy as jnp
import numpy as np
from jax.experimental import pallas as pl
from jax.experimental.pallas import tpu as pltpu

D_MODEL = 1024
N_HEADS = 16
D_HEAD = 64
N_EXP = 8
ROPE_BASE = 10000.0
NEG_INF = -1e30
HD = N_HEADS * D_HEAD
EHD = N_EXP * HD
EH = N_EXP * N_HEADS


def _top2_gates(logits_planes):
    m1 = logits_planes[0]
    i1 = jnp.zeros_like(m1)
    for e in range(1, N_EXP):
        gt = logits_planes[e] > m1
        m1 = jnp.where(gt, logits_planes[e], m1)
        i1 = jnp.where(gt, float(e), i1)
    p0 = jnp.where(i1 == 0.0, NEG_INF, logits_planes[0])
    m2 = p0
    i2 = jnp.zeros_like(m1)
    for e in range(1, N_EXP):
        pe = jnp.where(i1 == float(e), NEG_INF, logits_planes[e])
        gt = pe > m2
        m2 = jnp.where(gt, pe, m2)
        i2 = jnp.where(gt, float(e), i2)
    g1 = jax.nn.sigmoid(m1)
    g2 = jax.nn.sigmoid(m2)
    return [jnp.where(i1 == float(e), g1, 0.0) + jnp.where(i2 == float(e), g2, 0.0)
            for e in range(N_EXP)]


def _dot(a, b, trans_b=False):
    dims = (((1,), (1 if trans_b else 0,)), ((), ()))
    return jax.lax.dot_general(a, b, dims, preferred_element_type=jnp.float32)


def _swap_halves_lanes(t):
    r1 = pltpu.roll(t, HD - 32, 1)
    r2 = pltpu.roll(t, 32, 1)
    lane = jax.lax.broadcasted_iota(jnp.int32, t.shape, 1)
    return jnp.where(lane % D_HEAD < D_HEAD // 2, r1, r2)


def _proj_kernel(x_ref, wq4_ref, wv_ref, wsel_ref, rep_ref, cos_ref, sin_ref,
                 q_ref, k_ref, v_ref, go_ref):
    x = x_ref[...]
    xb = x.astype(jnp.bfloat16)

    qk4 = _dot(xb, wq4_ref[...])
    C = cos_ref[...]
    Sg = sin_ref[...]
    qh = qk4[:, :HD]
    kh = qk4[:, HD:]
    q_ref[...] = (qh * C + _swap_halves_lanes(qh) * Sg).astype(jnp.bfloat16)
    k_ref[...] = (kh * C + _swap_halves_lanes(kh) * Sg).astype(jnp.bfloat16)

    lg = _dot(x, wsel_ref[...])
    planes_v = [lg[:, e * N_HEADS:(e + 1) * N_HEADS] for e in range(N_EXP)]
    planes_o = [lg[:, EH + e * N_HEADS:EH + (e + 1) * N_HEADS] for e in range(N_EXP)]
    gates_v = _top2_gates(planes_v)
    gates_o = _top2_gates(planes_o)
    go_ref[...] = jnp.concatenate(gates_o, axis=1)
    gvb = jnp.concatenate(gates_v, axis=1).astype(jnp.bfloat16)

    vacc = None
    for e in range(N_EXP):
        ge = _dot(gvb, rep_ref[:, e * HD:(e + 1) * HD])
        ve = _dot(xb, wv_ref[:, e * HD:(e + 1) * HD])
        term = ve * ge
        vacc = term if vacc is None else vacc + term
    v_ref[...] = vacc.astype(jnp.bfloat16)


def _attn_kernel(q_ref, k_ref, v_ref, go_ref, rep_ref, wo_ref, o_ref, *, tq, tk):
    qt = pl.program_id(0)
    dh = D_HEAD
    rows = jax.lax.broadcasted_iota(jnp.int32, (tq, tk), 0) + qt * tq
    cols0 = jax.lax.broadcasted_iota(jnp.int32, (tq, tk), 1)
    ones = jnp.ones((tk, dh), jnp.bfloat16)
    m0 = jnp.full((tq, 1), NEG_INF, jnp.float32)
    a0 = jnp.zeros((tq, 2 * dh), jnp.float32)

    def step(q_, h, j, m, accl, masked):
        kj = k_ref[pl.ds(j * tk, tk), h * dh:(h + 1) * dh]
        vj = v_ref[pl.ds(j * tk, tk), h * dh:(h + 1) * dh]
        s = _dot(q_, kj, trans_b=True)
        if masked:
            s = jnp.where(cols0 + j * tk <= rows, s, NEG_INF)
        mn = jnp.maximum(m, jnp.max(s, axis=-1, keepdims=True))
        alpha = jnp.exp(m - mn)
        p = jnp.exp(s - mn)
        vjx = jnp.concatenate([vj, ones], axis=1)
        accl2 = accl * alpha + _dot(p.astype(jnp.bfloat16), vjx)
        return mn, accl2

    ctx_list = [None] * N_HEADS
    for h0 in range(0, N_HEADS, 2):
        qa = q_ref[:, h0 * dh:(h0 + 1) * dh]
        qb = q_ref[:, (h0 + 1) * dh:(h0 + 2) * dh]

        def body(j, carry, h0=h0, qa=qa, qb=qb):
            ma, accla, mb, acclb = carry
            ma, accla = step(qa, h0, j, ma, accla, masked=False)
            mb, acclb = step(qb, h0 + 1, j, mb, acclb, masked=False)
            return ma, accla, mb, acclb

        ma, accla, mb, acclb = jax.lax.fori_loop(0, qt, body, (m0, a0, m0, a0))
        ma, accla = step(qa, h0, qt, ma, accla, masked=True)
        mb, acclb = step(qb, h0 + 1, qt, mb, acclb, masked=True)
        ctx_list[h0] = accla[:, :dh] / accla[:, dh:dh + 1]
        ctx_list[h0 + 1] = acclb[:, :dh] / acclb[:, dh:dh + 1]
    ctx = jnp.concatenate(ctx_list, axis=1)
    gob = go_ref[...].astype(jnp.bfloat16)
    acc = None
    for e in range(N_EXP):
        ge = _dot(gob, rep_ref[:, e * HD:(e + 1) * HD])
        blk = (ctx * ge).astype(jnp.bfloat16)
        pe = _dot(blk, wo_ref[e * HD:(e + 1) * HD, :])
        acc = pe if acc is None else acc + pe
    o_ref[...] = acc


def kernel(token_stream, Wq, Wk, Wv, Wo, Wsel_v, Wsel_o):
    B, S, _ = token_stream.shape
    x = token_stream.reshape(S, D_MODEL)
    H, E, dh = N_HEADS, N_EXP, D_HEAD
    half = dh // 2
    f32 = jnp.float32
    bf16 = jnp.bfloat16

    scale = 1.0 / np.sqrt(dh)
    wq4 = jnp.concatenate([Wq * scale, Wk], axis=1).astype(bf16)
    wv_em = Wv.transpose(2, 1, 0, 3).reshape(D_MODEL, EHD).astype(bf16)
    wo_em = Wo.transpose(1, 0, 2, 3).reshape(EHD, D_MODEL).astype(bf16)
    wsel = jnp.concatenate(
        [Wsel_v.reshape(D_MODEL, H, E).transpose(0, 2, 1).reshape(D_MODEL, EH),
         Wsel_o.reshape(D_MODEL, H, E).transpose(0, 2, 1).reshape(D_MODEL, EH)],
        axis=1)

    rcols = jnp.arange(EHD)
    e_c = rcols // HD
    h_c = (rcols % HD) // dh
    rep = (jnp.arange(EH)[:, None] == (e_c * H + h_c)[None, :]).astype(bf16)

    pos = jnp.arange(S, dtype=f32)
    inv_freq = 1.0 / (ROPE_BASE ** (jnp.arange(0, dh, 2, dtype=f32) / dh))
    ang = pos[:, None] * inv_freq[None, :]
    chead = jnp.concatenate([jnp.cos(ang), jnp.cos(ang)], axis=1)
    shead = jnp.concatenate([-jnp.sin(ang), jnp.sin(ang)], axis=1)
    ctab = jnp.tile(chead, (1, H))
    stab = jnp.tile(shead, (1, H))

    T = min(256, S)
    nt = S // T

    q, k, v, go = pl.pallas_call(
        _proj_kernel,
        grid=(nt,),
        in_specs=[
            pl.BlockSpec((T, D_MODEL), lambda t: (t, 0)),
            pl.BlockSpec((D_MODEL, 2 * HD), lambda t: (0, 0)),
            pl.BlockSpec((D_MODEL, EHD), lambda t: (0, 0)),
            pl.BlockSpec((D_MODEL, 2 * EH), lambda t: (0, 0)),
            pl.BlockSpec((EH, EHD), lambda t: (0, 0)),
            pl.BlockSpec((T, HD), lambda t: (t, 0)),
            pl.BlockSpec((T, HD), lambda t: (t, 0)),
        ],
        out_specs=[
            pl.BlockSpec((T, HD), lambda t: (t, 0)),
            pl.BlockSpec((T, HD), lambda t: (t, 0)),
            pl.BlockSpec((T, HD), lambda t: (t, 0)),
            pl.BlockSpec((T, EH), lambda t: (t, 0)),
        ],
        out_shape=[
            jax.ShapeDtypeStruct((S, HD), bf16),
            jax.ShapeDtypeStruct((S, HD), bf16),
            jax.ShapeDtypeStruct((S, HD), bf16),
            jax.ShapeDtypeStruct((S, EH), f32),
        ],
    )(x, wq4, wv_em, wsel, rep, ctab, stab)

    Tq = min(512, S)
    Tk = min(512, S)
    nq = S // Tq
    out = pl.pallas_call(
        functools.partial(_attn_kernel, tq=Tq, tk=Tk),
        grid=(nq,),
        in_specs=[
            pl.BlockSpec((Tq, HD), lambda qt: (qt, 0)),
            pl.BlockSpec((S, HD), lambda qt: (0, 0)),
            pl.BlockSpec((S, HD), lambda qt: (0, 0)),
            pl.BlockSpec((Tq, EH), lambda qt: (qt, 0)),
            pl.BlockSpec((EH, EHD), lambda qt: (0, 0)),
            pl.BlockSpec((EHD, D_MODEL), lambda qt: (0, 0)),
        ],
        out_specs=pl.BlockSpec((Tq, D_MODEL), lambda qt: (qt, 0)),
        out_shape=jax.ShapeDtypeStruct((S, D_MODEL), f32),
    )(q, k, v, go, rep, wo_em)

    return out.reshape(B, S, D_MODEL)
